# baseline (speedup 1.0000x reference)
import functools

import jax
import jax.numpy as jnp
from jax import lax
from jax.experimental import pallas as pl
from jax.experimental.pallas import tpu as pltpu
from jax.experimental.pallas import tpu_sc as plsc

_B = 64
_HW = 65536
_OBJS = 256
_PARTS = 17
_NIDX = _OBJS * _PARTS

_LANES = 16
_NVEC = _NIDX // _LANES

@functools.cache
def _make_sc_gather():
    info = plsc.get_sparse_core_info()
    nc, ns = info.num_cores, info.num_subcores
    nw = nc * ns
    bpw = _B // nw
    mesh = plsc.VectorSubcoreMesh(core_axis_name="c", subcore_axis_name="s")

    @functools.partial(
        pl.kernel,
        mesh=mesh,
        compiler_params=pltpu.CompilerParams(needs_layout_passes=False),
        out_type=jax.ShapeDtypeStruct((_B, _PARTS, _OBJS), jnp.float32),
        scratch_types=[
            pltpu.VMEM((256, 256), jnp.float32),
            pltpu.VMEM((_OBJS, _PARTS), jnp.int32),
            pltpu.VMEM((_PARTS, _OBJS), jnp.float32),
            pltpu.SemaphoreType.DMA,
        ],
    )
    def sc_gather(ae_hbm, ind_hbm, out_hbm, img_v, idx_v, vals_v, sem):
        wid = lax.axis_index("s") * nc + lax.axis_index("c")
        for i in range(bpw):
            b = wid * bpw + i
            cp = pltpu.async_copy(ae_hbm.at[b], img_v, sem)
            pltpu.sync_copy(ind_hbm.at[b], idx_v)
            cp.wait()

            @plsc.parallel_loop(0, _NVEC, unroll=8)
            def _(k):
                p = k >> 4
                o0 = (k & 15) * _LANES
                rows = lax.iota(jnp.int32, _LANES) + o0
                idx = plsc.load_gather(idx_v, [rows, jnp.zeros((_LANES,), jnp.int32) + p])
                vals_v[p, pl.ds(o0, _LANES)] = plsc.load_gather(
                    img_v, [idx >> 8, idx & 255]
                )
            pltpu.sync_copy(vals_v, out_hbm.at[b])

    return sc_gather


_BB = 32


def _tc_body(tag_ref, mask_ref, pull_ref, push_ref):
    step = pl.program_id(0)
    t = tag_ref[...]
    m = mask_ref[...]
    s0 = jnp.sum(m, axis=1, keepdims=True)
    s1 = jnp.sum(t * m, axis=1, keepdims=True)
    tm = s1 / (s0 + 1e-4)
    has_obj = s0 != 0.0
    obj_num = jnp.sum(has_obj.astype(jnp.float32), axis=2, keepdims=True)

    pull_dist = (t - tm) ** 2 * m
    pull_num = jnp.sum(pull_dist, axis=(1, 2), keepdims=True)
    pull_b = pull_num / (obj_num + 1e-4)

    sent = 100.0 + 2.0 * lax.broadcasted_iota(
        jnp.int32, (_BB, 1, _OBJS), 2
    ).astype(jnp.float32)
    tmz = jnp.where(has_obj, tm, sent)
    ones = jnp.ones((_BB, 1, _OBJS), jnp.float32)
    dn = (((1,), (1,)), ((0,), (0,)))
    tmi = lax.dot_general(tmz, ones, dn, preferred_element_type=jnp.float32)
    tmj = jnp.broadcast_to(tmz, (_BB, _OBJS, _OBJS))
    pd = jnp.maximum(1.0 - jnp.abs(tmi - tmj), 0.0)
    push_sum = jnp.sum(pd, axis=(1, 2), keepdims=True)
    push_b = (push_sum - float(_OBJS)) / (obj_num * (obj_num - 1.0) + 1e-4)

    pull_v = jnp.sum(pull_b) / _B
    push_v = jnp.sum(push_b) / _B

    @pl.when(step == 0)
    def _():
        pull_ref[0, 0] = 0.0
        push_ref[0, 0] = 0.0

    pull_ref[0, 0] += pull_v
    push_ref[0, 0] += push_v


def _tc_loss(tag3, mask3):
    steps = _B // _BB
    return pl.pallas_call(
        _tc_body,
        grid=(steps,),
        in_specs=[
            pl.BlockSpec((_BB, _PARTS, _OBJS), lambda i: (i, 0, 0)),
            pl.BlockSpec((_BB, _PARTS, _OBJS), lambda i: (i, 0, 0)),
        ],
        out_specs=[
            pl.BlockSpec((1, 1), lambda i: (0, 0), memory_space=pltpu.SMEM),
            pl.BlockSpec((1, 1), lambda i: (0, 0), memory_space=pltpu.SMEM),
        ],
        out_shape=[
            jax.ShapeDtypeStruct((1, 1), jnp.float32),
            jax.ShapeDtypeStruct((1, 1), jnp.float32),
        ],
    )(tag3, mask3)


def kernel(ae, ind, ind_mask):
    b, _, h, w = ae.shape
    ae3 = ae.reshape(b, h, w)
    ind3 = ind.astype(jnp.int32)
    mask_pm = ind_mask.transpose(0, 2, 1)
    tag3 = _make_sc_gather()(ae3, ind3)
    pull, push = _tc_loss(tag3, mask_pm)
    return pull[0, 0], push[0, 0]


# speedup vs baseline: 1.3131x; 1.3131x over previous
import functools

import jax
import jax.numpy as jnp
from jax import lax
from jax.experimental import pallas as pl
from jax.experimental.pallas import tpu as pltpu
from jax.experimental.pallas import tpu_sc as plsc

_B = 64
_HW = 65536
_OBJS = 256
_PARTS = 17
_NIDX = _OBJS * _PARTS

_LANES = 16
_NVEC = _NIDX // _LANES

@functools.cache
def _make_sc_gather():
    info = plsc.get_sparse_core_info()
    nc, ns = info.num_cores, info.num_subcores
    nw = nc * ns
    bpw = _B // nw
    mesh = plsc.VectorSubcoreMesh(core_axis_name="c", subcore_axis_name="s")

    @functools.partial(
        pl.kernel,
        mesh=mesh,
        compiler_params=pltpu.CompilerParams(needs_layout_passes=False),
        out_type=jax.ShapeDtypeStruct((_B, _PARTS, _OBJS), jnp.float32),
        scratch_types=[
            pltpu.VMEM((256, 256), jnp.float32),
            pltpu.VMEM((_NIDX,), jnp.int32),
            pltpu.VMEM((_PARTS, _OBJS), jnp.float32),
            pltpu.SemaphoreType.DMA,
        ],
    )
    def sc_gather(ae_hbm, ind_hbm, out_hbm, img_v, idx_v, vals_v, sem):
        wid = lax.axis_index("s") * nc + lax.axis_index("c")
        for i in range(bpw):
            b = wid * bpw + i
            cp = pltpu.async_copy(ae_hbm.at[b], img_v, sem)
            pltpu.sync_copy(ind_hbm.at[b], idx_v)
            cp.wait()

            @plsc.parallel_loop(0, _NVEC, unroll=8)
            def _(k):
                idx = idx_v[pl.ds(k * _LANES, _LANES)]
                p = k >> 4
                o0 = (k & 15) * _LANES
                vals_v[p, pl.ds(o0, _LANES)] = plsc.load_gather(
                    img_v, [idx >> 8, idx & 255]
                )
            pltpu.sync_copy(vals_v, out_hbm.at[b])

    return sc_gather


_BB = 32


def _tc_body(tag_ref, mask_ref, pull_ref, push_ref):
    step = pl.program_id(0)
    t = tag_ref[...]
    m = mask_ref[...]
    s0 = jnp.sum(m, axis=1, keepdims=True)
    s1 = jnp.sum(t * m, axis=1, keepdims=True)
    tm = s1 / (s0 + 1e-4)
    has_obj = s0 != 0.0
    obj_num = jnp.sum(has_obj.astype(jnp.float32), axis=2, keepdims=True)

    pull_dist = (t - tm) ** 2 * m
    pull_num = jnp.sum(pull_dist, axis=(1, 2), keepdims=True)
    pull_b = pull_num / (obj_num + 1e-4)

    sent = 100.0 + 2.0 * lax.broadcasted_iota(
        jnp.int32, (_BB, 1, _OBJS), 2
    ).astype(jnp.float32)
    tmz = jnp.where(has_obj, tm, sent)
    ones = jnp.ones((_BB, 1, _OBJS), jnp.float32)
    dn = (((1,), (1,)), ((0,), (0,)))
    tmi = lax.dot_general(tmz, ones, dn, preferred_element_type=jnp.float32)
    tmj = jnp.broadcast_to(tmz, (_BB, _OBJS, _OBJS))
    pd = jnp.maximum(1.0 - jnp.abs(tmi - tmj), 0.0)
    push_sum = jnp.sum(pd, axis=(1, 2), keepdims=True)
    push_b = (push_sum - float(_OBJS)) / (obj_num * (obj_num - 1.0) + 1e-4)

    pull_v = jnp.sum(pull_b) / _B
    push_v = jnp.sum(push_b) / _B

    @pl.when(step == 0)
    def _():
        pull_ref[0, 0] = 0.0
        push_ref[0, 0] = 0.0

    pull_ref[0, 0] += pull_v
    push_ref[0, 0] += push_v


def _tc_loss(tag3, mask3):
    steps = _B // _BB
    return pl.pallas_call(
        _tc_body,
        grid=(steps,),
        in_specs=[
            pl.BlockSpec((_BB, _PARTS, _OBJS), lambda i: (i, 0, 0)),
            pl.BlockSpec((_BB, _PARTS, _OBJS), lambda i: (i, 0, 0)),
        ],
        out_specs=[
            pl.BlockSpec((1, 1), lambda i: (0, 0), memory_space=pltpu.SMEM),
            pl.BlockSpec((1, 1), lambda i: (0, 0), memory_space=pltpu.SMEM),
        ],
        out_shape=[
            jax.ShapeDtypeStruct((1, 1), jnp.float32),
            jax.ShapeDtypeStruct((1, 1), jnp.float32),
        ],
    )(tag3, mask3)


def kernel(ae, ind, ind_mask):
    b, _, h, w = ae.shape
    ae3 = ae.reshape(b, h, w)
    ind_pm = ind.transpose(0, 2, 1).reshape(b, _NIDX).astype(jnp.int32)
    mask_pm = ind_mask.transpose(0, 2, 1)
    tag3 = _make_sc_gather()(ae3, ind_pm)
    pull, push = _tc_loss(tag3, mask_pm)
    return pull[0, 0], push[0, 0]
